# initial kernel scaffold (unmeasured)
import jax
import jax.numpy as jnp
from jax import lax
from jax.experimental import pallas as pl
from jax.experimental.pallas import tpu as pltpu

N_DEV = 4
M_GLOBAL = 4096
N_FULL = 8192
M_CHUNK = M_GLOBAL // N_DEV
N_Q = N_FULL // 4
N_STRIP = 1024
FP8_MAX = 448.0


def _compiler_params(**kw):
    cp = getattr(pltpu, "CompilerParams", None) or pltpu.TPUCompilerParams
    return cp(**kw)


def kernel(x, w_mat):
    xb = x.astype(jnp.bfloat16)
    wb = w_mat.astype(jnp.bfloat16)
    partial = jnp.dot(xb, wb, preferred_element_type=jnp.float32)

    def body(p_ref, out_ref, comm_ref, local_ref, amax_ref,
             send_sems, recv_sems, load_sem, asend_sems, arecv_sems):
        my = lax.axis_index("i")
        left = (my - 1) % N_DEV
        right = (my + 1) % N_DEV

        barrier = pltpu.get_barrier_semaphore()
        for nbr in (left, right):
            pl.semaphore_signal(barrier, inc=1, device_id=(nbr,),
                                device_id_type=pl.DeviceIdType.MESH)
        pl.semaphore_wait(barrier, 2)

        def load_local(chunk, q):
            cp = pltpu.make_async_copy(
                p_ref.at[pl.ds(chunk * M_CHUNK, M_CHUNK),
                         pl.ds(q * N_Q, N_Q)],
                local_ref, load_sem)
            cp.start()
            cp.wait()

        for q in range(N_FULL // N_Q):
            cols = pl.ds(q * N_Q, N_Q)
            for s in range(N_DEV - 1):
                c = (my - s - 1) % N_DEV
                load_local(c, q)
                ss, rs = s % 2, (s + 1) % 2
                if s == 0:
                    comm_ref[ss] = local_ref[...].astype(jnp.bfloat16)
                else:
                    comm_ref[ss] = (comm_ref[ss].astype(jnp.float32)
                                    + local_ref[...]).astype(jnp.bfloat16)
                rdma = pltpu.make_async_remote_copy(
                    src_ref=comm_ref.at[ss], dst_ref=comm_ref.at[rs],
                    send_sem=send_sems.at[ss], recv_sem=recv_sems.at[rs],
                    device_id=(right,), device_id_type=pl.DeviceIdType.MESH)
                rdma.start()
                rdma.wait()
            load_local(my, q)
            out_ref[:, cols] = (comm_ref[(N_DEV - 1) % 2].astype(jnp.float32)
                                + local_ref[...])

        gl = jnp.float32(0.0)
        for strip in range(N_FULL // N_STRIP):
            sl = pl.ds(strip * N_STRIP, N_STRIP)
            gl = jnp.maximum(gl, jnp.max(jnp.abs(out_ref[:, sl])))
        amax_ref[0] = jnp.full((8, 128), gl, jnp.float32)
        rdmas = []
        for rel in range(1, N_DEV):
            tgt = (my - rel) % N_DEV
            r = pltpu.make_async_remote_copy(
                src_ref=amax_ref.at[0], dst_ref=amax_ref.at[rel],
                send_sem=asend_sems.at[rel], recv_sem=arecv_sems.at[rel],
                device_id=(tgt,), device_id_type=pl.DeviceIdType.MESH)
            r.start()
            rdmas.append(r)
        for r in rdmas:
            r.wait()
        gmax = jnp.max(amax_ref[...])

        scale = gmax / FP8_MAX
        inv_scale = FP8_MAX / gmax
        for strip in range(N_FULL // N_STRIP):
            sl = pl.ds(strip * N_STRIP, N_STRIP)
            v = jnp.clip(out_ref[:, sl] * inv_scale, -FP8_MAX, FP8_MAX)
            qv = v.astype(jnp.float8_e4m3fn)
            out_ref[:, sl] = qv.astype(jnp.float32) * scale

    return pl.pallas_call(
        body,
        out_shape=jax.ShapeDtypeStruct((M_CHUNK, N_FULL), jnp.float32),
        in_specs=[pl.BlockSpec(memory_space=pltpu.ANY)],
        out_specs=pl.BlockSpec(memory_space=pltpu.VMEM),
        scratch_shapes=[
            pltpu.VMEM((2, M_CHUNK, N_Q), jnp.bfloat16),
            pltpu.VMEM((M_CHUNK, N_Q), jnp.float32),
            pltpu.VMEM((N_DEV, 8, 128), jnp.float32),
            pltpu.SemaphoreType.DMA((2,)),
            pltpu.SemaphoreType.DMA((2,)),
            pltpu.SemaphoreType.DMA,
            pltpu.SemaphoreType.DMA((N_DEV,)),
            pltpu.SemaphoreType.DMA((N_DEV,)),
        ],
        compiler_params=_compiler_params(collective_id=0),
    )(partial)


# baseline (device time: 784686 ns/iter reference)
import jax
import jax.numpy as jnp
from jax import lax
from jax.experimental import pallas as pl
from jax.experimental.pallas import tpu as pltpu

N_DEV = 4
M_GLOBAL = 4096
N_FULL = 8192
M_CHUNK = M_GLOBAL // N_DEV
N_Q = N_FULL // 4
N_STRIP = 1024
FP8_MAX = 448.0


def _compiler_params(**kw):
    cp = getattr(pltpu, "CompilerParams", None) or pltpu.TPUCompilerParams
    return cp(**kw)


def kernel(x, w_mat):
    xb = x.astype(jnp.bfloat16)
    wb = w_mat.astype(jnp.bfloat16)
    partial = jnp.dot(xb, wb, preferred_element_type=jnp.float32)

    def body(p_ref, out_ref, comm_ref, local_ref, amax_ref,
             send_sems, recv_sems, load_sem, asend_sems, arecv_sems):
        my = lax.axis_index("i")
        left = (my - 1) % N_DEV
        right = (my + 1) % N_DEV

        barrier = pltpu.get_barrier_semaphore()
        for nbr in (left, right):
            pl.semaphore_signal(barrier, inc=1, device_id=(nbr,),
                                device_id_type=pl.DeviceIdType.MESH)
        pl.semaphore_wait(barrier, 2)

        def load_local(chunk, q):
            cp = pltpu.make_async_copy(
                p_ref.at[pl.ds(chunk * M_CHUNK, M_CHUNK),
                         pl.ds(q * N_Q, N_Q)],
                local_ref, load_sem)
            cp.start()
            cp.wait()

        for q in range(N_FULL // N_Q):
            cols = pl.ds(q * N_Q, N_Q)
            for s in range(N_DEV - 1):
                c = (my - s - 1) % N_DEV
                load_local(c, q)
                ss, rs = s % 2, (s + 1) % 2
                if s == 0:
                    comm_ref[ss] = local_ref[...].astype(jnp.bfloat16)
                else:
                    comm_ref[ss] = (comm_ref[ss].astype(jnp.float32)
                                    + local_ref[...]).astype(jnp.bfloat16)
                rdma = pltpu.make_async_remote_copy(
                    src_ref=comm_ref.at[ss], dst_ref=comm_ref.at[rs],
                    send_sem=send_sems.at[ss], recv_sem=recv_sems.at[rs],
                    device_id=(right,), device_id_type=pl.DeviceIdType.MESH)
                rdma.start()
                rdma.wait()
            load_local(my, q)
            out_ref[:, cols] = (comm_ref[(N_DEV - 1) % 2].astype(jnp.float32)
                                + local_ref[...])

        gl = jnp.float32(0.0)
        for strip in range(N_FULL // N_STRIP):
            sl = pl.ds(strip * N_STRIP, N_STRIP)
            gl = jnp.maximum(gl, jnp.max(jnp.abs(out_ref[:, sl])))
        amax_ref[0] = jnp.full((8, 128), gl, jnp.float32)
        rdmas = []
        for rel in range(1, N_DEV):
            tgt = (my - rel) % N_DEV
            r = pltpu.make_async_remote_copy(
                src_ref=amax_ref.at[0], dst_ref=amax_ref.at[rel],
                send_sem=asend_sems.at[rel], recv_sem=arecv_sems.at[rel],
                device_id=(tgt,), device_id_type=pl.DeviceIdType.MESH)
            r.start()
            rdmas.append(r)
        for r in rdmas:
            r.wait()
        gmax = jnp.max(amax_ref[...])

        scale = gmax / FP8_MAX
        inv_scale = FP8_MAX / gmax
        for strip in range(N_FULL // N_STRIP):
            sl = pl.ds(strip * N_STRIP, N_STRIP)
            v = jnp.clip(out_ref[:, sl] * inv_scale, -FP8_MAX, FP8_MAX)
            qv = v.astype(jnp.float8_e4m3fn)
            out_ref[:, sl] = qv.astype(jnp.float32) * scale

    return pl.pallas_call(
        body,
        out_shape=jax.ShapeDtypeStruct((M_CHUNK, N_FULL), jnp.float32),
        in_specs=[pl.BlockSpec(memory_space=pl.ANY)],
        out_specs=pl.BlockSpec(memory_space=pltpu.VMEM),
        scratch_shapes=[
            pltpu.VMEM((2, M_CHUNK, N_Q), jnp.bfloat16),
            pltpu.VMEM((M_CHUNK, N_Q), jnp.float32),
            pltpu.VMEM((N_DEV, 8, 128), jnp.float32),
            pltpu.SemaphoreType.DMA((2,)),
            pltpu.SemaphoreType.DMA((2,)),
            pltpu.SemaphoreType.DMA,
            pltpu.SemaphoreType.DMA((N_DEV,)),
            pltpu.SemaphoreType.DMA((N_DEV,)),
        ],
        compiler_params=_compiler_params(
            collective_id=0, vmem_limit_bytes=63 * 1024 * 1024),
    )(partial)


# device time: 531587 ns/iter; 1.4761x vs baseline; 1.4761x over previous
import jax
import jax.numpy as jnp
from jax import lax
from jax.experimental import pallas as pl
from jax.experimental.pallas import tpu as pltpu

N_DEV = 4
M_GLOBAL = 4096
N_FULL = 8192
M_CHUNK = M_GLOBAL // N_DEV
N_Q = N_FULL // 4
FP8_MAX = 448.0


def _compiler_params(**kw):
    cp = getattr(pltpu, "CompilerParams", None) or pltpu.TPUCompilerParams
    return cp(**kw)


def kernel(x, w_mat):
    xb = x.astype(jnp.bfloat16)
    wb = w_mat.astype(jnp.bfloat16)
    partial = jnp.dot(xb, wb, preferred_element_type=jnp.float32)

    def body(p_ref, out_ref, comm_r, comm_l, local_ref, amax_ref,
             send_r, recv_r, send_l, recv_l, load_sem, store_sem,
             asend_sems, arecv_sems):
        my = lax.axis_index("i")
        left = (my - 1) % N_DEV
        right = (my + 1) % N_DEV

        barrier = pltpu.get_barrier_semaphore()
        for nbr in (left, right):
            pl.semaphore_signal(barrier, inc=1, device_id=(nbr,),
                                device_id_type=pl.DeviceIdType.MESH)
        pl.semaphore_wait(barrier, 2)

        def load_local(chunk, q):
            cp = pltpu.make_async_copy(
                p_ref.at[pl.ds(chunk * M_CHUNK, M_CHUNK),
                         pl.ds(q * N_Q, N_Q)],
                local_ref, load_sem)
            cp.start()
            cp.wait()

        def ring_rdma(comm, sends, recvs, s, nbr):
            ss, rs = s % 2, (s + 1) % 2
            return pltpu.make_async_remote_copy(
                src_ref=comm.at[ss], dst_ref=comm.at[rs],
                send_sem=sends.at[ss], recv_sem=recvs.at[rs],
                device_id=(nbr,), device_id_type=pl.DeviceIdType.MESH)

        def accum_send(comm, s, chunk, q):
            load_local(chunk, q)
            ss = s % 2
            if s == 0:
                comm[ss] = local_ref[...].astype(jnp.bfloat16)
            else:
                comm[ss] = (comm[ss].astype(jnp.float32)
                            + local_ref[...]).astype(jnp.bfloat16)

        def final_quarter(comm, q):
            load_local(my, q)
            local_ref[...] = (comm[(N_DEV - 1) % 2].astype(jnp.float32)
                              + local_ref[...])
            part_amax = jnp.max(jnp.abs(local_ref[...]))
            st = pltpu.make_async_copy(
                local_ref, out_ref.at[:, pl.ds(q * N_Q, N_Q)], store_sem)
            st.start()
            st.wait()
            return part_amax

        gl = jnp.float32(0.0)
        for qp in range(2):
            for s in range(N_DEV - 1):
                accum_send(comm_r, s, (my - s - 1) % N_DEV, qp)
                rdma_r = ring_rdma(comm_r, send_r, recv_r, s, right)
                rdma_r.start()
                accum_send(comm_l, s, (my + s + 1) % N_DEV, qp + 2)
                rdma_l = ring_rdma(comm_l, send_l, recv_l, s, left)
                rdma_l.start()
                rdma_r.wait()
                rdma_l.wait()
            gl = jnp.maximum(gl, final_quarter(comm_r, qp))
            gl = jnp.maximum(gl, final_quarter(comm_l, qp + 2))

        amax_ref[0] = jnp.full((8, 128), gl, jnp.float32)
        rdmas = []
        for rel in range(1, N_DEV):
            tgt = (my - rel) % N_DEV
            r = pltpu.make_async_remote_copy(
                src_ref=amax_ref.at[0], dst_ref=amax_ref.at[rel],
                send_sem=asend_sems.at[rel], recv_sem=arecv_sems.at[rel],
                device_id=(tgt,), device_id_type=pl.DeviceIdType.MESH)
            r.start()
            rdmas.append(r)
        for r in rdmas:
            r.wait()
        gmax = jnp.max(amax_ref[...])

        scale = gmax / FP8_MAX
        inv_scale = FP8_MAX / gmax
        for q in range(4):
            cols = pl.ds(q * N_Q, N_Q)
            ld = pltpu.make_async_copy(out_ref.at[:, cols], local_ref,
                                       load_sem)
            ld.start()
            ld.wait()
            v = jnp.clip(local_ref[...] * inv_scale, -FP8_MAX, FP8_MAX)
            qv = v.astype(jnp.float8_e4m3fn)
            local_ref[...] = qv.astype(jnp.float32) * scale
            st = pltpu.make_async_copy(local_ref, out_ref.at[:, cols],
                                       store_sem)
            st.start()
            st.wait()

    return pl.pallas_call(
        body,
        out_shape=jax.ShapeDtypeStruct((M_CHUNK, N_FULL), jnp.float32),
        in_specs=[pl.BlockSpec(memory_space=pl.ANY)],
        out_specs=pl.BlockSpec(memory_space=pl.ANY),
        scratch_shapes=[
            pltpu.VMEM((2, M_CHUNK, N_Q), jnp.bfloat16),
            pltpu.VMEM((2, M_CHUNK, N_Q), jnp.bfloat16),
            pltpu.VMEM((M_CHUNK, N_Q), jnp.float32),
            pltpu.VMEM((N_DEV, 8, 128), jnp.float32),
            pltpu.SemaphoreType.DMA((2,)),
            pltpu.SemaphoreType.DMA((2,)),
            pltpu.SemaphoreType.DMA((2,)),
            pltpu.SemaphoreType.DMA((2,)),
            pltpu.SemaphoreType.DMA,
            pltpu.SemaphoreType.DMA,
            pltpu.SemaphoreType.DMA((N_DEV,)),
            pltpu.SemaphoreType.DMA((N_DEV,)),
        ],
        compiler_params=_compiler_params(
            collective_id=0, vmem_limit_bytes=63 * 1024 * 1024),
    )(partial)


# device time: 395728 ns/iter; 1.9829x vs baseline; 1.3433x over previous
import jax
import jax.numpy as jnp
from jax import lax
from jax.experimental import pallas as pl
from jax.experimental.pallas import tpu as pltpu

N_DEV = 4
M_GLOBAL = 4096
K_SHARD = 1024
N_FULL = 8192
M_CHUNK = M_GLOBAL // N_DEV
N_Q = N_FULL // 4
FP8_MAX = 448.0


def _compiler_params(**kw):
    cp = getattr(pltpu, "CompilerParams", None) or pltpu.TPUCompilerParams
    return cp(**kw)


def kernel(x, w_mat):
    xb = x.astype(jnp.bfloat16)
    wb = w_mat.astype(jnp.bfloat16)

    def body(x_ref, w_ref, out_ref, wbuf, comm_r, comm_l, gr, glb, amax_ref,
             send_r, recv_r, send_l, recv_l, wsem_r, wsem_l,
             store_r, store_l, asend_sems, arecv_sems):
        my = lax.axis_index("i")
        left = (my - 1) % N_DEV
        right = (my + 1) % N_DEV

        def load_w(q, slot, sem):
            cp = pltpu.make_async_copy(
                w_ref.at[:, pl.ds(q * N_Q, N_Q)], wbuf.at[slot], sem)
            cp.start()
            return cp

        def gemm(chunk, slot):
            return jnp.dot(x_ref[pl.ds(chunk * M_CHUNK, M_CHUNK), :],
                           wbuf[slot], preferred_element_type=jnp.float32)

        def ring_rdma(comm, sends, recvs, s, nbr):
            ss, rs = s % 2, (s + 1) % 2
            return pltpu.make_async_remote_copy(
                src_ref=comm.at[ss], dst_ref=comm.at[rs],
                send_sem=sends.at[ss], recv_sem=recvs.at[rs],
                device_id=(nbr,), device_id_type=pl.DeviceIdType.MESH)

        def store_out(src, q, sem):
            cp = pltpu.make_async_copy(
                src, out_ref.at[:, pl.ds(q * N_Q, N_Q)], sem)
            cp.start()
            return cp

        wr = load_w(0, 0, wsem_r)
        wl = load_w(2, 1, wsem_l)

        barrier = pltpu.get_barrier_semaphore()
        for nbr in (left, right):
            pl.semaphore_signal(barrier, inc=1, device_id=(nbr,),
                                device_id_type=pl.DeviceIdType.MESH)
        pl.semaphore_wait(barrier, 2)

        wr.wait()
        wl.wait()
        comm_r[0] = gemm((my - 1) % N_DEV, 0).astype(jnp.bfloat16)
        comm_l[0] = gemm((my + 1) % N_DEV, 1).astype(jnp.bfloat16)

        gl_amax = jnp.float32(0.0)
        for qp in range(2):
            for s in range(N_DEV - 1):
                ss = s % 2
                if s > 0:
                    comm_r[ss] = (comm_r[ss].astype(jnp.float32)
                                  + gr[...]).astype(jnp.bfloat16)
                    comm_l[ss] = (comm_l[ss].astype(jnp.float32)
                                  + glb[...]).astype(jnp.bfloat16)
                rdma_r = ring_rdma(comm_r, send_r, recv_r, s, right)
                rdma_r.start()
                rdma_l = ring_rdma(comm_l, send_l, recv_l, s, left)
                rdma_l.start()
                if s < N_DEV - 2:
                    gr[...] = gemm((my - s - 2) % N_DEV, 0)
                    glb[...] = gemm((my + s + 2) % N_DEV, 1)
                else:
                    gr[...] = gemm(my, 0)
                    glb[...] = gemm(my, 1)
                    if qp == 0:
                        wr = load_w(1, 0, wsem_r)
                        wl = load_w(3, 1, wsem_l)
                rdma_r.wait()
                rdma_l.wait()
            fs = (N_DEV - 1) % 2
            gr[...] = comm_r[fs].astype(jnp.float32) + gr[...]
            glb[...] = comm_l[fs].astype(jnp.float32) + glb[...]
            gl_amax = jnp.maximum(gl_amax, jnp.max(jnp.abs(gr[...])))
            gl_amax = jnp.maximum(gl_amax, jnp.max(jnp.abs(glb[...])))
            if qp == 0:
                st_r = store_out(gr, 0, store_r)
                st_l = store_out(glb, 2, store_l)
                wr.wait()
                wl.wait()
                comm_r[0] = gemm((my - 1) % N_DEV, 0).astype(jnp.bfloat16)
                comm_l[0] = gemm((my + 1) % N_DEV, 1).astype(jnp.bfloat16)
                st_r.wait()
                st_l.wait()

        amax_ref[0] = jnp.full((8, 128), gl_amax, jnp.float32)
        rdmas = []
        for rel in range(1, N_DEV):
            tgt = (my - rel) % N_DEV
            r = pltpu.make_async_remote_copy(
                src_ref=amax_ref.at[0], dst_ref=amax_ref.at[rel],
                send_sem=asend_sems.at[rel], recv_sem=arecv_sems.at[rel],
                device_id=(tgt,), device_id_type=pl.DeviceIdType.MESH)
            r.start()
            rdmas.append(r)
        for r in rdmas:
            r.wait()
        gmax = jnp.max(amax_ref[...])

        scale = gmax / FP8_MAX
        inv_scale = FP8_MAX / gmax

        def quant(buf):
            v = jnp.clip(buf[...] * inv_scale, -FP8_MAX, FP8_MAX)
            qv = v.astype(jnp.float8_e4m3fn)
            buf[...] = qv.astype(jnp.float32) * scale

        quant(gr)
        st_r = store_out(gr, 1, store_r)
        quant(glb)
        st_l = store_out(glb, 3, store_l)
        st_r.wait()
        ld0 = pltpu.make_async_copy(out_ref.at[:, pl.ds(0, N_Q)], gr, wsem_r)
        ld0.start()
        st_l.wait()
        ld2 = pltpu.make_async_copy(out_ref.at[:, pl.ds(2 * N_Q, N_Q)], glb,
                                    wsem_l)
        ld2.start()
        ld0.wait()
        quant(gr)
        st_r = store_out(gr, 0, store_r)
        ld2.wait()
        quant(glb)
        st_l = store_out(glb, 2, store_l)
        st_r.wait()
        st_l.wait()

    return pl.pallas_call(
        body,
        out_shape=jax.ShapeDtypeStruct((M_CHUNK, N_FULL), jnp.float32),
        in_specs=[pl.BlockSpec(memory_space=pltpu.VMEM),
                  pl.BlockSpec(memory_space=pl.ANY)],
        out_specs=pl.BlockSpec(memory_space=pl.ANY),
        scratch_shapes=[
            pltpu.VMEM((2, K_SHARD, N_Q), jnp.bfloat16),
            pltpu.VMEM((2, M_CHUNK, N_Q), jnp.bfloat16),
            pltpu.VMEM((2, M_CHUNK, N_Q), jnp.bfloat16),
            pltpu.VMEM((M_CHUNK, N_Q), jnp.float32),
            pltpu.VMEM((M_CHUNK, N_Q), jnp.float32),
            pltpu.VMEM((N_DEV, 8, 128), jnp.float32),
            pltpu.SemaphoreType.DMA((2,)),
            pltpu.SemaphoreType.DMA((2,)),
            pltpu.SemaphoreType.DMA((2,)),
            pltpu.SemaphoreType.DMA((2,)),
            pltpu.SemaphoreType.DMA,
            pltpu.SemaphoreType.DMA,
            pltpu.SemaphoreType.DMA,
            pltpu.SemaphoreType.DMA,
            pltpu.SemaphoreType.DMA((N_DEV,)),
            pltpu.SemaphoreType.DMA((N_DEV,)),
        ],
        compiler_params=_compiler_params(
            collective_id=0, vmem_limit_bytes=63 * 1024 * 1024),
    )(xb, wb)


# device time: 393711 ns/iter; 1.9931x vs baseline; 1.0051x over previous
import jax
import jax.numpy as jnp
from jax import lax
from jax.experimental import pallas as pl
from jax.experimental.pallas import tpu as pltpu

N_DEV = 4
M_GLOBAL = 4096
K_SHARD = 1024
N_FULL = 8192
M_CHUNK = M_GLOBAL // N_DEV
N_Q = N_FULL // 4
FP8_MAX = 448.0


def _compiler_params(**kw):
    cp = getattr(pltpu, "CompilerParams", None) or pltpu.TPUCompilerParams
    return cp(**kw)


def kernel(x, w_mat):
    xb = x.astype(jnp.bfloat16)
    wb = w_mat.astype(jnp.bfloat16)

    def body(x_ref, w_ref, out_ref, wbuf, comm_r, comm_l, gr, glb, amax_ref,
             send_r, recv_r, send_l, recv_l, wsem_r, wsem_l,
             store_r, store_l, asend_sems, arecv_sems):
        my = lax.axis_index("i")
        left = (my - 1) % N_DEV
        right = (my + 1) % N_DEV

        def load_w(q, slot, sem):
            cp = pltpu.make_async_copy(
                w_ref.at[:, pl.ds(q * N_Q, N_Q)], wbuf.at[slot], sem)
            cp.start()
            return cp

        def gemm(chunk, slot):
            return jnp.dot(x_ref[pl.ds(chunk * M_CHUNK, M_CHUNK), :],
                           wbuf[slot], preferred_element_type=jnp.float32)

        def ring_rdma(comm, sends, recvs, s, nbr):
            ss, rs = s % 2, (s + 1) % 2
            return pltpu.make_async_remote_copy(
                src_ref=comm.at[ss], dst_ref=comm.at[rs],
                send_sem=sends.at[ss], recv_sem=recvs.at[rs],
                device_id=(nbr,), device_id_type=pl.DeviceIdType.MESH)

        def store_out(src, q, sem):
            cp = pltpu.make_async_copy(
                src, out_ref.at[:, pl.ds(q * N_Q, N_Q)], sem)
            cp.start()
            return cp

        wr = load_w(0, 0, wsem_r)
        wl = load_w(2, 1, wsem_l)

        barrier = pltpu.get_barrier_semaphore()
        for nbr in (left, right):
            pl.semaphore_signal(barrier, inc=1, device_id=(nbr,),
                                device_id_type=pl.DeviceIdType.MESH)
        pl.semaphore_wait(barrier, 2)

        def fold(comm, slot, buf):
            comm[slot] = (comm[slot].astype(jnp.float32)
                          + buf[...]).astype(jnp.bfloat16)

        wr.wait()
        wl.wait()
        comm_r[0] = gemm((my - 1) % N_DEV, 0).astype(jnp.bfloat16)
        rr = ring_rdma(comm_r, send_r, recv_r, 0, right)
        rr.start()
        comm_l[0] = gemm((my + 1) % N_DEV, 1).astype(jnp.bfloat16)
        rl = ring_rdma(comm_l, send_l, recv_l, 0, left)
        rl.start()

        gl_amax = jnp.float32(0.0)
        for qp in range(2):
            if qp == 1:
                gl_amax = jnp.maximum(gl_amax, jnp.max(jnp.abs(gr[...])))
                st_r = store_out(gr, 0, store_r)
                gl_amax = jnp.maximum(gl_amax, jnp.max(jnp.abs(glb[...])))
                st_l = store_out(glb, 2, store_l)
                st_r.wait()
                st_l.wait()
            gr[...] = gemm((my - 2) % N_DEV, 0)
            glb[...] = gemm((my + 2) % N_DEV, 1)
            rr.wait()
            rl.wait()
            fold(comm_r, 1, gr)
            rr = ring_rdma(comm_r, send_r, recv_r, 1, right)
            rr.start()
            fold(comm_l, 1, glb)
            rl = ring_rdma(comm_l, send_l, recv_l, 1, left)
            rl.start()
            gr[...] = gemm((my - 3) % N_DEV, 0)
            glb[...] = gemm((my + 3) % N_DEV, 1)
            rr.wait()
            rl.wait()
            fold(comm_r, 0, gr)
            rr = ring_rdma(comm_r, send_r, recv_r, 2, right)
            rr.start()
            fold(comm_l, 0, glb)
            rl = ring_rdma(comm_l, send_l, recv_l, 2, left)
            rl.start()
            gr[...] = gemm(my, 0)
            glb[...] = gemm(my, 1)
            if qp == 0:
                wr = load_w(1, 0, wsem_r)
                wl = load_w(3, 1, wsem_l)
            rr.wait()
            rl.wait()
            if qp == 0:
                gr[...] = comm_r[1].astype(jnp.float32) + gr[...]
                glb[...] = comm_l[1].astype(jnp.float32) + glb[...]
                wr.wait()
                wl.wait()
                comm_r[0] = gemm((my - 1) % N_DEV, 0).astype(jnp.bfloat16)
                rr = ring_rdma(comm_r, send_r, recv_r, 0, right)
                rr.start()
                comm_l[0] = gemm((my + 1) % N_DEV, 1).astype(jnp.bfloat16)
                rl = ring_rdma(comm_l, send_l, recv_l, 0, left)
                rl.start()
        gr[...] = comm_r[1].astype(jnp.float32) + gr[...]
        glb[...] = comm_l[1].astype(jnp.float32) + glb[...]
        gl_amax = jnp.maximum(gl_amax, jnp.max(jnp.abs(gr[...])))
        gl_amax = jnp.maximum(gl_amax, jnp.max(jnp.abs(glb[...])))

        amax_ref[0] = jnp.full((8, 128), gl_amax, jnp.float32)
        rdmas = []
        for rel in range(1, N_DEV):
            tgt = (my - rel) % N_DEV
            r = pltpu.make_async_remote_copy(
                src_ref=amax_ref.at[0], dst_ref=amax_ref.at[rel],
                send_sem=asend_sems.at[rel], recv_sem=arecv_sems.at[rel],
                device_id=(tgt,), device_id_type=pl.DeviceIdType.MESH)
            r.start()
            rdmas.append(r)
        for r in rdmas:
            r.wait()
        gmax = jnp.max(amax_ref[...])

        scale = gmax / FP8_MAX
        inv_scale = FP8_MAX / gmax

        def quant(buf):
            v = jnp.clip(buf[...] * inv_scale, -FP8_MAX, FP8_MAX)
            qv = v.astype(jnp.float8_e4m3fn)
            buf[...] = qv.astype(jnp.float32) * scale

        quant(gr)
        st_r = store_out(gr, 1, store_r)
        quant(glb)
        st_l = store_out(glb, 3, store_l)
        st_r.wait()
        ld0 = pltpu.make_async_copy(out_ref.at[:, pl.ds(0, N_Q)], gr, wsem_r)
        ld0.start()
        st_l.wait()
        ld2 = pltpu.make_async_copy(out_ref.at[:, pl.ds(2 * N_Q, N_Q)], glb,
                                    wsem_l)
        ld2.start()
        ld0.wait()
        quant(gr)
        st_r = store_out(gr, 0, store_r)
        ld2.wait()
        quant(glb)
        st_l = store_out(glb, 2, store_l)
        st_r.wait()
        st_l.wait()

    return pl.pallas_call(
        body,
        out_shape=jax.ShapeDtypeStruct((M_CHUNK, N_FULL), jnp.float32),
        in_specs=[pl.BlockSpec(memory_space=pltpu.VMEM),
                  pl.BlockSpec(memory_space=pl.ANY)],
        out_specs=pl.BlockSpec(memory_space=pl.ANY),
        scratch_shapes=[
            pltpu.VMEM((2, K_SHARD, N_Q), jnp.bfloat16),
            pltpu.VMEM((2, M_CHUNK, N_Q), jnp.bfloat16),
            pltpu.VMEM((2, M_CHUNK, N_Q), jnp.bfloat16),
            pltpu.VMEM((M_CHUNK, N_Q), jnp.float32),
            pltpu.VMEM((M_CHUNK, N_Q), jnp.float32),
            pltpu.VMEM((N_DEV, 8, 128), jnp.float32),
            pltpu.SemaphoreType.DMA((2,)),
            pltpu.SemaphoreType.DMA((2,)),
            pltpu.SemaphoreType.DMA((2,)),
            pltpu.SemaphoreType.DMA((2,)),
            pltpu.SemaphoreType.DMA,
            pltpu.SemaphoreType.DMA,
            pltpu.SemaphoreType.DMA,
            pltpu.SemaphoreType.DMA,
            pltpu.SemaphoreType.DMA((N_DEV,)),
            pltpu.SemaphoreType.DMA((N_DEV,)),
        ],
        compiler_params=_compiler_params(
            collective_id=0, vmem_limit_bytes=63 * 1024 * 1024),
    )(xb, wb)
